# baseline (speedup 1.0000x reference)
import functools

import jax
import jax.numpy as jnp
from jax.experimental import pallas as pl
from jax.experimental.pallas import tpu as pltpu

_MATCH_THRESH = 0.5
_NEG_POS = 3.0
_V0 = 0.1
_V1 = 0.2


def _smooth_l1(x):
    ax = jnp.abs(x)
    return jnp.where(ax < 1.0, 0.5 * ax * ax, ax - 0.5)


def _loss_kernel(tgt_ref, priors_ref, obj_ref, rloc_ref, ploc_ref, pconf_ref,
                 out_ref, bto_s, bti_s, pos1_s, omask_s, conf2_s, pos2_s, neg_s,
                 *, A, C, TP, R):
    b = pl.program_id(0)

    @pl.when(b == 0)
    def _init():
        out_ref[...] = jnp.zeros_like(out_ref)

    f32 = jnp.float32
    lane = jax.lax.broadcasted_iota(jnp.int32, (1, 128), 1).astype(f32)
    t_col = jax.lax.broadcasted_iota(jnp.int32, (TP, 1), 0).astype(f32)

    tg = tgt_ref[0]
    tx1 = tg[:, 0:1]
    ty1 = tg[:, 1:2]
    tx2 = tg[:, 2:3]
    ty2 = tg[:, 3:4]
    tlab = tg[:, 4:5]
    area_t = (tx2 - tx1) * (ty2 - ty1)

    NCH = 8

    def anchor_block(j4, refined):
        sl = pl.ds(j4, NCH)
        cx = priors_ref[0, sl, :]
        cy = priors_ref[1, sl, :]
        w = priors_ref[2, sl, :]
        h = priors_ref[3, sl, :]
        if refined:
            l0 = rloc_ref[0, 0, sl, :]
            l1 = rloc_ref[0, 1, sl, :]
            l2 = rloc_ref[0, 2, sl, :]
            l3 = rloc_ref[0, 3, sl, :]
            cx = cx + l0 * (_V0 * w)
            cy = cy + l1 * (_V0 * h)
            w = w * jnp.exp(l2 * _V1)
            h = h * jnp.exp(l3 * _V1)
        return cx, cy, w, h

    def overlaps_row(cx, cy, w, h, kk):
        cxk = cx[kk:kk + 1, :]
        cyk = cy[kk:kk + 1, :]
        wk = w[kk:kk + 1, :]
        hk = h[kk:kk + 1, :]
        ax1 = cxk - 0.5 * wk
        ay1 = cyk - 0.5 * hk
        ax2 = cxk + 0.5 * wk
        ay2 = cyk + 0.5 * hk
        iw = jnp.clip(jnp.minimum(tx2, ax2) - jnp.maximum(tx1, ax1),
                      0.0, None)
        ih = jnp.clip(jnp.minimum(ty2, ay2) - jnp.maximum(ty1, ay1),
                      0.0, None)
        inter = iw * ih
        area_a = (ax2 - ax1) * (ay2 - ay1)
        return inter / (area_t + area_a - inter)

    def pass_a(refined):
        def body(jj, carry):
            run_max, run_idx = carry
            j4 = jj * NCH
            cx, cy, w, h = anchor_block(j4, refined)
            base = jnp.float32(128.0) * j4.astype(f32) + lane
            btos, btis = [], []
            m_acc = i_acc = None
            for kk in range(0, NCH, 2):
                ov_a = overlaps_row(cx, cy, w, h, kk)
                ov_b = overlaps_row(cx, cy, w, h, kk + 1)
                for ov in (ov_a, ov_b):
                    btos.append(jnp.max(ov, axis=0, keepdims=True))
                    btis.append(jnp.min(jnp.where(ov == btos[-1], t_col, 1e9),
                                        axis=0, keepdims=True))
                m = jnp.maximum(ov_a, ov_b)
                i = jnp.where(ov_b > ov_a, base + jnp.float32(128.0 * (kk + 1)),
                              base + jnp.float32(128.0 * kk))
                if m_acc is None:
                    m_acc, i_acc = m, i
                else:
                    i_acc = jnp.where(m > m_acc, i, i_acc)
                    m_acc = jnp.maximum(m_acc, m)
            bto_s[pl.ds(j4, NCH), :] = jnp.concatenate(btos, axis=0)
            bti_s[pl.ds(j4, NCH), :] = jnp.concatenate(btis, axis=0)
            m, i = m_acc, i_acc
            better = m > run_max
            run_idx = jnp.where(better, i, run_idx)
            run_max = jnp.maximum(run_max, m)
            return run_max, run_idx

        init = (jnp.full((TP, 128), -1e9, f32), jnp.full((TP, 128), 1e9, f32))
        run_max, run_idx = jax.lax.fori_loop(0, R // NCH, body, init)
        gmax = jnp.max(run_max, axis=1, keepdims=True)
        bpidx = jnp.min(jnp.where(run_max == gmax, run_idx, 1e9),
                        axis=1, keepdims=True)
        return jnp.where(t_col < 50.0, bpidx, -1.0)

    def pass_b(refined, bpidx, loc_pred_ref, store_blk, masked):
        sub4 = jax.lax.broadcasted_iota(jnp.int32, (NCH, 128), 0).astype(f32)
        lane4 = jax.lax.broadcasted_iota(jnp.int32, (NCH, 128), 1).astype(f32)

        def body(jj, carry):
            acc_n, acc_l = carry
            j4 = jj * NCH
            sl = pl.ds(j4, NCH)
            bti_u = bti_s[sl, :]
            bto_u = bto_s[sl, :]
            rows = []
            for kk in range(NCH):
                ga = jnp.float32(128.0) * (j4.astype(f32) +
                                           jnp.float32(kk)) + lane
                eqf = bpidx == ga
                forced_t = jnp.max(jnp.where(eqf, t_col, -1.0), axis=0,
                                   keepdims=True)
                hasf = forced_t >= 0.0
                bti_f = jnp.where(hasf, forced_t, bti_u[kk:kk + 1, :])
                bto_f = jnp.where(hasf, 2.0, bto_u[kk:kk + 1, :])
                oh = t_col == bti_f
                gsum = lambda v: jnp.sum(jnp.where(oh, v, 0.0), axis=0,
                                         keepdims=True)
                rows.append((gsum(tlab), gsum(tx1), gsum(ty1), gsum(tx2),
                             gsum(ty2), bto_f))
            cat = lambda q: jnp.concatenate([r[q] for r in rows], axis=0)
            conf_g = cat(0)
            mx1 = cat(1)
            my1 = cat(2)
            mx2 = cat(3)
            my2 = cat(4)
            bto4 = cat(5)
            ga4 = (jnp.float32(128.0) * j4.astype(f32) + sub4 * 128.0 + lane4)
            conf = jnp.where(bto4 < _MATCH_THRESH, 0.0, conf_g)
            posf = jnp.where((conf > 0.0) & (ga4 < jnp.float32(A)), 1.0, 0.0)
            if masked:
                posf = posf * omask_s[sl, :]
            cx, cy, w, h = anchor_block(j4, refined)
            ecx = ((mx1 + mx2) * 0.5 - cx) / (_V0 * w)
            ecy = ((my1 + my2) * 0.5 - cy) / (_V0 * h)
            ew = jnp.log(jnp.maximum((mx2 - mx1) / w, 1e-8)) / _V1
            eh = jnp.log(jnp.maximum((my2 - my1) / h, 1e-8)) / _V1
            p0 = loc_pred_ref[0, 0, sl, :]
            p1 = loc_pred_ref[0, 1, sl, :]
            p2 = loc_pred_ref[0, 2, sl, :]
            p3 = loc_pred_ref[0, 3, sl, :]
            ll = (_smooth_l1(p0 - ecx) + _smooth_l1(p1 - ecy) +
                  _smooth_l1(p2 - ew) + _smooth_l1(p3 - eh))
            store_blk(sl, conf, posf)
            return acc_n + posf, acc_l + ll * posf

        init = (jnp.zeros((NCH, 128), f32), jnp.zeros((NCH, 128), f32))
        return jax.lax.fori_loop(0, R // NCH, body, init)

    bpidx1 = pass_a(False)

    def store1(sl, conf, posf):
        pos1_s[sl, :] = posf

    n1_blk, l1_blk = pass_b(False, bpidx1, rloc_ref, store1, False)
    out_ref[0:1, :] += jnp.sum(n1_blk, axis=0, keepdims=True)
    out_ref[1:2, :] += jnp.sum(l1_blk, axis=0, keepdims=True)

    NG = R // 8
    idx8 = (jax.lax.broadcasted_iota(jnp.int32, (8, 128), 0) * 128 +
            jax.lax.broadcasted_iota(jnp.int32, (8, 128), 1)).astype(f32)
    log_theta = jnp.log(jnp.float32(0.99))

    def obj_stage(g, acc):
        sl = pl.ds(g * 8, 8)
        valid = (idx8 + jnp.float32(1024.0) * g.astype(f32)) < jnp.float32(A)
        o0 = obj_ref[0, 0, sl, :]
        o1 = obj_ref[0, 1, sl, :]
        m = jnp.maximum(o0, o1)
        lse2 = m + jnp.log(jnp.exp(o0 - m) + jnp.exp(o1 - m))
        pos1 = pos1_s[sl, :]
        ce2 = lse2 - (o0 * (1.0 - pos1) + o1 * pos1)
        omask_s[sl, :] = jnp.where((o0 - lse2) < log_theta, 1.0, 0.0)
        return acc + jnp.where(valid, ce2, 0.0)

    acc2 = jax.lax.fori_loop(0, NG, obj_stage, jnp.zeros((8, 128), f32))
    out_ref[2:3, :] += jnp.sum(acc2, axis=0, keepdims=True)

    bpidx2 = pass_a(True)

    def store2(sl, conf, posf):
        conf2_s[sl, :] = conf
        pos2_s[sl, :] = posf

    n2_blk, l2_blk = pass_b(True, bpidx2, ploc_ref, store2, True)
    out_ref[3:4, :] += jnp.sum(n2_blk, axis=0, keepdims=True)
    out_ref[4:5, :] += jnp.sum(l2_blk, axis=0, keepdims=True)

    def ce_stage(g, carry):
        acc5, accn = carry
        sl = pl.ds(g * 8, 8)
        valid = (idx8 + jnp.float32(1024.0) * g.astype(f32)) < jnp.float32(A)
        maxp = pconf_ref[0, 0, sl, :]
        for c in range(1, C):
            maxp = jnp.maximum(maxp, pconf_ref[0, c, sl, :])
        conf2 = conf2_s[sl, :]
        s = jnp.zeros((8, 128), f32)
        picked = jnp.zeros((8, 128), f32)
        for c in range(C):
            pc = pconf_ref[0, c, sl, :]
            s = s + jnp.exp(pc - maxp)
            picked = picked + jnp.where(conf2 == jnp.float32(c), pc, 0.0)
        ce_all = maxp + jnp.log(s) - picked
        pos2 = pos2_s[sl, :]
        neg_s[sl, :] = jnp.where((conf2 == 0.0) & (omask_s[sl, :] > 0.5) &
                                 valid, ce_all, -1.0)
        return acc5 + ce_all * pos2, accn + pos2

    (acc5, accn) = jax.lax.fori_loop(
        0, NG, ce_stage, (jnp.zeros((8, 128), f32), jnp.zeros((8, 128), f32)))
    out_ref[5:6, :] += jnp.sum(acc5, axis=0, keepdims=True)

    pos_num = jnp.sum(accn)
    k = jnp.maximum(10.0, jnp.minimum(pos_num * _NEG_POS,
                                      jnp.float32(A) - pos_num))

    def count_sum(thr):
        def body(g, carry):
            cacc, sacc = carry
            ng = neg_s[pl.ds(g * 8, 8), :]
            above = ng > thr
            return (cacc + jnp.where(above, 1.0, 0.0),
                    sacc + jnp.where(above, ng, 0.0))

        cacc, sacc = jax.lax.fori_loop(
            0, NG, body, (jnp.zeros((8, 128), f32), jnp.zeros((8, 128), f32)))
        return jnp.sum(cacc), jnp.sum(sacc)

    def bis(_, carry):
        lo, hi = carry
        mid = 0.5 * (lo + hi)

        def body(g, cacc):
            return cacc + jnp.where(neg_s[pl.ds(g * 8, 8), :] > mid, 1.0, 0.0)

        cnt = jnp.sum(jax.lax.fori_loop(0, NG, body,
                                        jnp.zeros((8, 128), f32)))
        ge = cnt >= k
        return jnp.where(ge, mid, lo), jnp.where(ge, hi, mid)

    lo, hi = jax.lax.fori_loop(0, 32, bis, (jnp.float32(-2.0),
                                            jnp.float32(64.0)))
    c_gt, s_gt = count_sum(jnp.maximum(lo, -0.5))
    mined = s_gt + jnp.where(lo > -0.5, (k - c_gt) * lo, 0.0)
    out_ref[6:7, :] += jnp.where(lane < 1.0, mined, 0.0)


def kernel(objectness, refine_loc, pred_conf, pred_loc, anchors, targets):
    B, A, C = pred_conf.shape
    T = targets.shape[1]
    AP = ((A + 1023) // 1024) * 1024
    R = AP // 128
    TP = ((T + 7) // 8) * 8

    priors = anchors[0]
    pad = jnp.concatenate([jnp.full((AP - A, 2), -10.0, jnp.float32),
                           jnp.ones((AP - A, 2), jnp.float32)], axis=1)
    priors_pl = jnp.concatenate([priors, pad], axis=0).T.reshape(4, R, 128)

    def planes(x):
        k = x.shape[-1]
        xp = jnp.pad(x, ((0, 0), (0, AP - A), (0, 0)))
        return xp.transpose(0, 2, 1).reshape(B, k, R, 128)

    obj_pl = planes(objectness)
    rloc_pl = planes(refine_loc)
    ploc_pl = planes(pred_loc)
    pconf_pl = planes(pred_conf)
    tgt = jnp.pad(targets, ((0, 0), (0, TP - T), (0, 3)))

    krn = functools.partial(_loss_kernel, A=A, C=C, TP=TP, R=R)
    out = pl.pallas_call(
        krn,
        grid=(B,),
        in_specs=[
            pl.BlockSpec((1, TP, 8), lambda b: (b, 0, 0)),
            pl.BlockSpec((4, R, 128), lambda b: (0, 0, 0)),
            pl.BlockSpec((1, 2, R, 128), lambda b: (b, 0, 0, 0)),
            pl.BlockSpec((1, 4, R, 128), lambda b: (b, 0, 0, 0)),
            pl.BlockSpec((1, 4, R, 128), lambda b: (b, 0, 0, 0)),
            pl.BlockSpec((1, C, R, 128), lambda b: (b, 0, 0, 0)),
        ],
        out_specs=pl.BlockSpec((8, 128), lambda b: (0, 0)),
        out_shape=jax.ShapeDtypeStruct((8, 128), jnp.float32),
        scratch_shapes=[pltpu.VMEM((R, 128), jnp.float32) for _ in range(7)],
        compiler_params=pltpu.CompilerParams(
            dimension_semantics=("arbitrary",)),
    )(tgt, priors_pl, obj_pl, rloc_pl, ploc_pl, pconf_pl)

    sums = jnp.sum(out, axis=1)
    arm_n, arm_loc, arm_cls, n, loc, cls_pos, neg_sum = (
        sums[0], sums[1], sums[2], sums[3], sums[4], sums[5], sums[6])
    class_loss = (cls_pos + neg_sum) / n
    loc_loss = loc / n
    arm_cls_loss = 0.04 * arm_cls / arm_n
    arm_loc_loss = arm_loc / arm_n
    total = class_loss + loc_loss + arm_cls_loss + arm_loc_loss
    return (total, class_loss, loc_loss, arm_cls_loss, arm_loc_loss)


# speedup vs baseline: 1.0643x; 1.0643x over previous
import functools

import jax
import jax.numpy as jnp
from jax.experimental import pallas as pl
from jax.experimental.pallas import tpu as pltpu

_MATCH_THRESH = 0.5
_NEG_POS = 3.0
_V0 = 0.1
_V1 = 0.2


def _smooth_l1(x):
    ax = jnp.abs(x)
    return jnp.where(ax < 1.0, 0.5 * ax * ax, ax - 0.5)


def _loss_kernel(tgt_ref, priors_ref, obj_ref, rloc_ref, ploc_ref, pconf_ref,
                 out_ref, bto_s, bti_s, pos1_s, omask_s, conf2_s, pos2_s, neg_s,
                 *, A, C, TP, R):
    b = pl.program_id(0)

    @pl.when(b == 0)
    def _init():
        out_ref[...] = jnp.zeros_like(out_ref)

    f32 = jnp.float32
    lane = jax.lax.broadcasted_iota(jnp.int32, (1, 128), 1).astype(f32)
    t_col = jax.lax.broadcasted_iota(jnp.int32, (TP, 1), 0).astype(f32)

    tg = tgt_ref[0]
    tx1 = tg[:, 0:1]
    ty1 = tg[:, 1:2]
    tx2 = tg[:, 2:3]
    ty2 = tg[:, 3:4]
    tlab = tg[:, 4:5]
    area_t = (tx2 - tx1) * (ty2 - ty1)

    NCH = 16

    def anchor_block(j4, refined):
        sl = pl.ds(j4, NCH)
        cx = priors_ref[0, sl, :]
        cy = priors_ref[1, sl, :]
        w = priors_ref[2, sl, :]
        h = priors_ref[3, sl, :]
        if refined:
            l0 = rloc_ref[0, 0, sl, :]
            l1 = rloc_ref[0, 1, sl, :]
            l2 = rloc_ref[0, 2, sl, :]
            l3 = rloc_ref[0, 3, sl, :]
            cx = cx + l0 * (_V0 * w)
            cy = cy + l1 * (_V0 * h)
            w = w * jnp.exp(l2 * _V1)
            h = h * jnp.exp(l3 * _V1)
        return cx, cy, w, h

    def overlaps_row(cx, cy, w, h, kk):
        cxk = cx[kk:kk + 1, :]
        cyk = cy[kk:kk + 1, :]
        wk = w[kk:kk + 1, :]
        hk = h[kk:kk + 1, :]
        ax1 = cxk - 0.5 * wk
        ay1 = cyk - 0.5 * hk
        ax2 = cxk + 0.5 * wk
        ay2 = cyk + 0.5 * hk
        iw = jnp.clip(jnp.minimum(tx2, ax2) - jnp.maximum(tx1, ax1),
                      0.0, None)
        ih = jnp.clip(jnp.minimum(ty2, ay2) - jnp.maximum(ty1, ay1),
                      0.0, None)
        inter = iw * ih
        area_a = (ax2 - ax1) * (ay2 - ay1)
        return inter / (area_t + area_a - inter)

    def pass_a(refined):
        def body(jj, carry):
            run_max, run_idx = carry
            j4 = jj * NCH
            cx, cy, w, h = anchor_block(j4, refined)
            base = jnp.float32(128.0) * j4.astype(f32) + lane
            btos, btis = [], []
            m_acc = i_acc = None
            for kk in range(0, NCH, 2):
                ov_a = overlaps_row(cx, cy, w, h, kk)
                ov_b = overlaps_row(cx, cy, w, h, kk + 1)
                for ov in (ov_a, ov_b):
                    btos.append(jnp.max(ov, axis=0, keepdims=True))
                    btis.append(jnp.min(jnp.where(ov == btos[-1], t_col, 1e9),
                                        axis=0, keepdims=True))
                m = jnp.maximum(ov_a, ov_b)
                i = jnp.where(ov_b > ov_a, base + jnp.float32(128.0 * (kk + 1)),
                              base + jnp.float32(128.0 * kk))
                if m_acc is None:
                    m_acc, i_acc = m, i
                else:
                    i_acc = jnp.where(m > m_acc, i, i_acc)
                    m_acc = jnp.maximum(m_acc, m)
            bto_s[pl.ds(j4, NCH), :] = jnp.concatenate(btos, axis=0)
            bti_s[pl.ds(j4, NCH), :] = jnp.concatenate(btis, axis=0)
            m, i = m_acc, i_acc
            better = m > run_max
            run_idx = jnp.where(better, i, run_idx)
            run_max = jnp.maximum(run_max, m)
            return run_max, run_idx

        init = (jnp.full((TP, 128), -1e9, f32), jnp.full((TP, 128), 1e9, f32))
        run_max, run_idx = jax.lax.fori_loop(0, R // NCH, body, init)
        gmax = jnp.max(run_max, axis=1, keepdims=True)
        bpidx = jnp.min(jnp.where(run_max == gmax, run_idx, 1e9),
                        axis=1, keepdims=True)
        return jnp.where(t_col < 50.0, bpidx, -1.0)

    def pass_b(refined, bpidx, loc_pred_ref, store_blk, masked):
        sub4 = jax.lax.broadcasted_iota(jnp.int32, (NCH, 128), 0).astype(f32)
        lane4 = jax.lax.broadcasted_iota(jnp.int32, (NCH, 128), 1).astype(f32)

        def body(jj, carry):
            acc_n, acc_l = carry
            j4 = jj * NCH
            sl = pl.ds(j4, NCH)
            bti_u = bti_s[sl, :]
            bto_u = bto_s[sl, :]
            rows = []
            for kk in range(NCH):
                ga = jnp.float32(128.0) * (j4.astype(f32) +
                                           jnp.float32(kk)) + lane
                eqf = bpidx == ga
                forced_t = jnp.max(jnp.where(eqf, t_col, -1.0), axis=0,
                                   keepdims=True)
                hasf = forced_t >= 0.0
                bti_f = jnp.where(hasf, forced_t, bti_u[kk:kk + 1, :])
                bto_f = jnp.where(hasf, 2.0, bto_u[kk:kk + 1, :])
                oh = t_col == bti_f
                gsum = lambda v: jnp.sum(jnp.where(oh, v, 0.0), axis=0,
                                         keepdims=True)
                rows.append((gsum(tlab), gsum(tx1), gsum(ty1), gsum(tx2),
                             gsum(ty2), bto_f))
            cat = lambda q: jnp.concatenate([r[q] for r in rows], axis=0)
            conf_g = cat(0)
            mx1 = cat(1)
            my1 = cat(2)
            mx2 = cat(3)
            my2 = cat(4)
            bto4 = cat(5)
            ga4 = (jnp.float32(128.0) * j4.astype(f32) + sub4 * 128.0 + lane4)
            conf = jnp.where(bto4 < _MATCH_THRESH, 0.0, conf_g)
            posf = jnp.where((conf > 0.0) & (ga4 < jnp.float32(A)), 1.0, 0.0)
            if masked:
                posf = posf * omask_s[sl, :]
            cx, cy, w, h = anchor_block(j4, refined)
            ecx = ((mx1 + mx2) * 0.5 - cx) / (_V0 * w)
            ecy = ((my1 + my2) * 0.5 - cy) / (_V0 * h)
            ew = jnp.log(jnp.maximum((mx2 - mx1) / w, 1e-8)) / _V1
            eh = jnp.log(jnp.maximum((my2 - my1) / h, 1e-8)) / _V1
            p0 = loc_pred_ref[0, 0, sl, :]
            p1 = loc_pred_ref[0, 1, sl, :]
            p2 = loc_pred_ref[0, 2, sl, :]
            p3 = loc_pred_ref[0, 3, sl, :]
            ll = (_smooth_l1(p0 - ecx) + _smooth_l1(p1 - ecy) +
                  _smooth_l1(p2 - ew) + _smooth_l1(p3 - eh))
            store_blk(sl, conf, posf)
            return acc_n + posf, acc_l + ll * posf

        init = (jnp.zeros((NCH, 128), f32), jnp.zeros((NCH, 128), f32))
        return jax.lax.fori_loop(0, R // NCH, body, init)

    bpidx1 = pass_a(False)

    def store1(sl, conf, posf):
        pos1_s[sl, :] = posf

    n1_blk, l1_blk = pass_b(False, bpidx1, rloc_ref, store1, False)
    out_ref[0:1, :] += jnp.sum(n1_blk, axis=0, keepdims=True)
    out_ref[1:2, :] += jnp.sum(l1_blk, axis=0, keepdims=True)

    NG = R // 8
    idx8 = (jax.lax.broadcasted_iota(jnp.int32, (8, 128), 0) * 128 +
            jax.lax.broadcasted_iota(jnp.int32, (8, 128), 1)).astype(f32)
    log_theta = jnp.log(jnp.float32(0.99))

    def obj_stage(g, acc):
        sl = pl.ds(g * 8, 8)
        valid = (idx8 + jnp.float32(1024.0) * g.astype(f32)) < jnp.float32(A)
        o0 = obj_ref[0, 0, sl, :]
        o1 = obj_ref[0, 1, sl, :]
        m = jnp.maximum(o0, o1)
        lse2 = m + jnp.log(jnp.exp(o0 - m) + jnp.exp(o1 - m))
        pos1 = pos1_s[sl, :]
        ce2 = lse2 - (o0 * (1.0 - pos1) + o1 * pos1)
        omask_s[sl, :] = jnp.where((o0 - lse2) < log_theta, 1.0, 0.0)
        return acc + jnp.where(valid, ce2, 0.0)

    acc2 = jax.lax.fori_loop(0, NG, obj_stage, jnp.zeros((8, 128), f32))
    out_ref[2:3, :] += jnp.sum(acc2, axis=0, keepdims=True)

    bpidx2 = pass_a(True)

    def store2(sl, conf, posf):
        conf2_s[sl, :] = conf
        pos2_s[sl, :] = posf

    n2_blk, l2_blk = pass_b(True, bpidx2, ploc_ref, store2, True)
    out_ref[3:4, :] += jnp.sum(n2_blk, axis=0, keepdims=True)
    out_ref[4:5, :] += jnp.sum(l2_blk, axis=0, keepdims=True)

    def ce_stage(g, carry):
        acc5, accn = carry
        sl = pl.ds(g * 8, 8)
        valid = (idx8 + jnp.float32(1024.0) * g.astype(f32)) < jnp.float32(A)
        maxp = pconf_ref[0, 0, sl, :]
        for c in range(1, C):
            maxp = jnp.maximum(maxp, pconf_ref[0, c, sl, :])
        conf2 = conf2_s[sl, :]
        s = jnp.zeros((8, 128), f32)
        picked = jnp.zeros((8, 128), f32)
        for c in range(C):
            pc = pconf_ref[0, c, sl, :]
            s = s + jnp.exp(pc - maxp)
            picked = picked + jnp.where(conf2 == jnp.float32(c), pc, 0.0)
        ce_all = maxp + jnp.log(s) - picked
        pos2 = pos2_s[sl, :]
        neg_s[sl, :] = jnp.where((conf2 == 0.0) & (omask_s[sl, :] > 0.5) &
                                 valid, ce_all, -1.0)
        return acc5 + ce_all * pos2, accn + pos2

    (acc5, accn) = jax.lax.fori_loop(
        0, NG, ce_stage, (jnp.zeros((8, 128), f32), jnp.zeros((8, 128), f32)))
    out_ref[5:6, :] += jnp.sum(acc5, axis=0, keepdims=True)

    pos_num = jnp.sum(accn)
    k = jnp.maximum(10.0, jnp.minimum(pos_num * _NEG_POS,
                                      jnp.float32(A) - pos_num))

    def count_sum(thr):
        def body(g, carry):
            cacc, sacc = carry
            ng = neg_s[pl.ds(g * 8, 8), :]
            above = ng > thr
            return (cacc + jnp.where(above, 1.0, 0.0),
                    sacc + jnp.where(above, ng, 0.0))

        cacc, sacc = jax.lax.fori_loop(
            0, NG, body, (jnp.zeros((8, 128), f32), jnp.zeros((8, 128), f32)))
        return jnp.sum(cacc), jnp.sum(sacc)

    def bis(_, carry):
        lo, hi = carry
        mid = 0.5 * (lo + hi)

        def body(g, cacc):
            return cacc + jnp.where(neg_s[pl.ds(g * 8, 8), :] > mid, 1.0, 0.0)

        cnt = jnp.sum(jax.lax.fori_loop(0, NG, body,
                                        jnp.zeros((8, 128), f32)))
        ge = cnt >= k
        return jnp.where(ge, mid, lo), jnp.where(ge, hi, mid)

    lo, hi = jax.lax.fori_loop(0, 32, bis, (jnp.float32(-2.0),
                                            jnp.float32(64.0)))
    c_gt, s_gt = count_sum(jnp.maximum(lo, -0.5))
    mined = s_gt + jnp.where(lo > -0.5, (k - c_gt) * lo, 0.0)
    out_ref[6:7, :] += jnp.where(lane < 1.0, mined, 0.0)


def kernel(objectness, refine_loc, pred_conf, pred_loc, anchors, targets):
    B, A, C = pred_conf.shape
    T = targets.shape[1]
    AP = ((A + 1023) // 1024) * 1024
    R = AP // 128
    TP = ((T + 7) // 8) * 8

    priors = anchors[0]
    pad = jnp.concatenate([jnp.full((AP - A, 2), -10.0, jnp.float32),
                           jnp.ones((AP - A, 2), jnp.float32)], axis=1)
    priors_pl = jnp.concatenate([priors, pad], axis=0).T.reshape(4, R, 128)

    def planes(x):
        k = x.shape[-1]
        xp = jnp.pad(x, ((0, 0), (0, AP - A), (0, 0)))
        return xp.transpose(0, 2, 1).reshape(B, k, R, 128)

    obj_pl = planes(objectness)
    rloc_pl = planes(refine_loc)
    ploc_pl = planes(pred_loc)
    pconf_pl = planes(pred_conf)
    tgt = jnp.pad(targets, ((0, 0), (0, TP - T), (0, 3)))

    krn = functools.partial(_loss_kernel, A=A, C=C, TP=TP, R=R)
    out = pl.pallas_call(
        krn,
        grid=(B,),
        in_specs=[
            pl.BlockSpec((1, TP, 8), lambda b: (b, 0, 0)),
            pl.BlockSpec((4, R, 128), lambda b: (0, 0, 0)),
            pl.BlockSpec((1, 2, R, 128), lambda b: (b, 0, 0, 0)),
            pl.BlockSpec((1, 4, R, 128), lambda b: (b, 0, 0, 0)),
            pl.BlockSpec((1, 4, R, 128), lambda b: (b, 0, 0, 0)),
            pl.BlockSpec((1, C, R, 128), lambda b: (b, 0, 0, 0)),
        ],
        out_specs=pl.BlockSpec((8, 128), lambda b: (0, 0)),
        out_shape=jax.ShapeDtypeStruct((8, 128), jnp.float32),
        scratch_shapes=[pltpu.VMEM((R, 128), jnp.float32) for _ in range(7)],
        compiler_params=pltpu.CompilerParams(
            dimension_semantics=("arbitrary",)),
    )(tgt, priors_pl, obj_pl, rloc_pl, ploc_pl, pconf_pl)

    sums = jnp.sum(out, axis=1)
    arm_n, arm_loc, arm_cls, n, loc, cls_pos, neg_sum = (
        sums[0], sums[1], sums[2], sums[3], sums[4], sums[5], sums[6])
    class_loss = (cls_pos + neg_sum) / n
    loc_loss = loc / n
    arm_cls_loss = 0.04 * arm_cls / arm_n
    arm_loc_loss = arm_loc / arm_n
    total = class_loss + loc_loss + arm_cls_loss + arm_loc_loss
    return (total, class_loss, loc_loss, arm_cls_loss, arm_loc_loss)


# speedup vs baseline: 1.1247x; 1.0567x over previous
import functools

import jax
import jax.numpy as jnp
from jax.experimental import pallas as pl
from jax.experimental.pallas import tpu as pltpu

_MATCH_THRESH = 0.5
_NEG_POS = 3.0
_V0 = 0.1
_V1 = 0.2


def _smooth_l1(x):
    ax = jnp.abs(x)
    return jnp.where(ax < 1.0, 0.5 * ax * ax, ax - 0.5)


def _loss_kernel(tgt_ref, priors_ref, obj_ref, rloc_ref, ploc_ref, pconf_ref,
                 out_ref, bto_s, bti_s, pos1_s, omask_s, conf2_s, pos2_s, neg_s,
                 *, A, C, TP, R):
    b = pl.program_id(0)

    @pl.when(b == 0)
    def _init():
        out_ref[...] = jnp.zeros_like(out_ref)

    f32 = jnp.float32
    lane = jax.lax.broadcasted_iota(jnp.int32, (1, 128), 1).astype(f32)
    t_col = jax.lax.broadcasted_iota(jnp.int32, (TP, 1), 0).astype(f32)

    tg = tgt_ref[0]
    tx1 = tg[:, 0:1]
    ty1 = tg[:, 1:2]
    tx2 = tg[:, 2:3]
    ty2 = tg[:, 3:4]
    tlab = tg[:, 4:5]
    area_t = (tx2 - tx1) * (ty2 - ty1)

    NCH = 32

    def anchor_block(j4, refined):
        sl = pl.ds(j4, NCH)
        cx = priors_ref[0, sl, :]
        cy = priors_ref[1, sl, :]
        w = priors_ref[2, sl, :]
        h = priors_ref[3, sl, :]
        if refined:
            l0 = rloc_ref[0, 0, sl, :]
            l1 = rloc_ref[0, 1, sl, :]
            l2 = rloc_ref[0, 2, sl, :]
            l3 = rloc_ref[0, 3, sl, :]
            cx = cx + l0 * (_V0 * w)
            cy = cy + l1 * (_V0 * h)
            w = w * jnp.exp(l2 * _V1)
            h = h * jnp.exp(l3 * _V1)
        return cx, cy, w, h

    def overlaps_row(cx, cy, w, h, kk):
        cxk = cx[kk:kk + 1, :]
        cyk = cy[kk:kk + 1, :]
        wk = w[kk:kk + 1, :]
        hk = h[kk:kk + 1, :]
        ax1 = cxk - 0.5 * wk
        ay1 = cyk - 0.5 * hk
        ax2 = cxk + 0.5 * wk
        ay2 = cyk + 0.5 * hk
        iw = jnp.clip(jnp.minimum(tx2, ax2) - jnp.maximum(tx1, ax1),
                      0.0, None)
        ih = jnp.clip(jnp.minimum(ty2, ay2) - jnp.maximum(ty1, ay1),
                      0.0, None)
        inter = iw * ih
        area_a = (ax2 - ax1) * (ay2 - ay1)
        return inter / (area_t + area_a - inter)

    def pass_a(refined):
        def body(jj, carry):
            run_max, run_idx = carry
            j4 = jj * NCH
            cx, cy, w, h = anchor_block(j4, refined)
            base = jnp.float32(128.0) * j4.astype(f32) + lane
            btos, btis = [], []
            m_acc = i_acc = None
            for kk in range(0, NCH, 2):
                ov_a = overlaps_row(cx, cy, w, h, kk)
                ov_b = overlaps_row(cx, cy, w, h, kk + 1)
                for ov in (ov_a, ov_b):
                    btos.append(jnp.max(ov, axis=0, keepdims=True))
                    btis.append(jnp.min(jnp.where(ov == btos[-1], t_col, 1e9),
                                        axis=0, keepdims=True))
                m = jnp.maximum(ov_a, ov_b)
                i = jnp.where(ov_b > ov_a, base + jnp.float32(128.0 * (kk + 1)),
                              base + jnp.float32(128.0 * kk))
                if m_acc is None:
                    m_acc, i_acc = m, i
                else:
                    i_acc = jnp.where(m > m_acc, i, i_acc)
                    m_acc = jnp.maximum(m_acc, m)
            bto_s[pl.ds(j4, NCH), :] = jnp.concatenate(btos, axis=0)
            bti_s[pl.ds(j4, NCH), :] = jnp.concatenate(btis, axis=0)
            m, i = m_acc, i_acc
            better = m > run_max
            run_idx = jnp.where(better, i, run_idx)
            run_max = jnp.maximum(run_max, m)
            return run_max, run_idx

        init = (jnp.full((TP, 128), -1e9, f32), jnp.full((TP, 128), 1e9, f32))
        run_max, run_idx = jax.lax.fori_loop(0, R // NCH, body, init)
        gmax = jnp.max(run_max, axis=1, keepdims=True)
        bpidx = jnp.min(jnp.where(run_max == gmax, run_idx, 1e9),
                        axis=1, keepdims=True)
        return jnp.where(t_col < 50.0, bpidx, -1.0)

    def pass_b(refined, bpidx, loc_pred_ref, store_blk, masked):
        sub4 = jax.lax.broadcasted_iota(jnp.int32, (NCH, 128), 0).astype(f32)
        lane4 = jax.lax.broadcasted_iota(jnp.int32, (NCH, 128), 1).astype(f32)

        def body(jj, carry):
            acc_n, acc_l = carry
            j4 = jj * NCH
            sl = pl.ds(j4, NCH)
            bti_u = bti_s[sl, :]
            bto_u = bto_s[sl, :]
            rows = []
            for kk in range(NCH):
                ga = jnp.float32(128.0) * (j4.astype(f32) +
                                           jnp.float32(kk)) + lane
                eqf = bpidx == ga
                forced_t = jnp.max(jnp.where(eqf, t_col, -1.0), axis=0,
                                   keepdims=True)
                hasf = forced_t >= 0.0
                bti_f = jnp.where(hasf, forced_t, bti_u[kk:kk + 1, :])
                bto_f = jnp.where(hasf, 2.0, bto_u[kk:kk + 1, :])
                oh = t_col == bti_f
                gsum = lambda v: jnp.sum(jnp.where(oh, v, 0.0), axis=0,
                                         keepdims=True)
                rows.append((gsum(tlab), gsum(tx1), gsum(ty1), gsum(tx2),
                             gsum(ty2), bto_f))
            cat = lambda q: jnp.concatenate([r[q] for r in rows], axis=0)
            conf_g = cat(0)
            mx1 = cat(1)
            my1 = cat(2)
            mx2 = cat(3)
            my2 = cat(4)
            bto4 = cat(5)
            ga4 = (jnp.float32(128.0) * j4.astype(f32) + sub4 * 128.0 + lane4)
            conf = jnp.where(bto4 < _MATCH_THRESH, 0.0, conf_g)
            posf = jnp.where((conf > 0.0) & (ga4 < jnp.float32(A)), 1.0, 0.0)
            if masked:
                posf = posf * omask_s[sl, :]
            cx, cy, w, h = anchor_block(j4, refined)
            ecx = ((mx1 + mx2) * 0.5 - cx) / (_V0 * w)
            ecy = ((my1 + my2) * 0.5 - cy) / (_V0 * h)
            ew = jnp.log(jnp.maximum((mx2 - mx1) / w, 1e-8)) / _V1
            eh = jnp.log(jnp.maximum((my2 - my1) / h, 1e-8)) / _V1
            p0 = loc_pred_ref[0, 0, sl, :]
            p1 = loc_pred_ref[0, 1, sl, :]
            p2 = loc_pred_ref[0, 2, sl, :]
            p3 = loc_pred_ref[0, 3, sl, :]
            ll = (_smooth_l1(p0 - ecx) + _smooth_l1(p1 - ecy) +
                  _smooth_l1(p2 - ew) + _smooth_l1(p3 - eh))
            store_blk(sl, conf, posf)
            return acc_n + posf, acc_l + ll * posf

        init = (jnp.zeros((NCH, 128), f32), jnp.zeros((NCH, 128), f32))
        return jax.lax.fori_loop(0, R // NCH, body, init)

    bpidx1 = pass_a(False)

    def store1(sl, conf, posf):
        pos1_s[sl, :] = posf

    n1_blk, l1_blk = pass_b(False, bpidx1, rloc_ref, store1, False)
    out_ref[0:1, :] += jnp.sum(n1_blk, axis=0, keepdims=True)
    out_ref[1:2, :] += jnp.sum(l1_blk, axis=0, keepdims=True)

    NG = R // 8
    idx8 = (jax.lax.broadcasted_iota(jnp.int32, (8, 128), 0) * 128 +
            jax.lax.broadcasted_iota(jnp.int32, (8, 128), 1)).astype(f32)
    log_theta = jnp.log(jnp.float32(0.99))

    def obj_stage(g, acc):
        sl = pl.ds(g * 8, 8)
        valid = (idx8 + jnp.float32(1024.0) * g.astype(f32)) < jnp.float32(A)
        o0 = obj_ref[0, 0, sl, :]
        o1 = obj_ref[0, 1, sl, :]
        m = jnp.maximum(o0, o1)
        lse2 = m + jnp.log(jnp.exp(o0 - m) + jnp.exp(o1 - m))
        pos1 = pos1_s[sl, :]
        ce2 = lse2 - (o0 * (1.0 - pos1) + o1 * pos1)
        omask_s[sl, :] = jnp.where((o0 - lse2) < log_theta, 1.0, 0.0)
        return acc + jnp.where(valid, ce2, 0.0)

    acc2 = jax.lax.fori_loop(0, NG, obj_stage, jnp.zeros((8, 128), f32))
    out_ref[2:3, :] += jnp.sum(acc2, axis=0, keepdims=True)

    bpidx2 = pass_a(True)

    def store2(sl, conf, posf):
        conf2_s[sl, :] = conf
        pos2_s[sl, :] = posf

    n2_blk, l2_blk = pass_b(True, bpidx2, ploc_ref, store2, True)
    out_ref[3:4, :] += jnp.sum(n2_blk, axis=0, keepdims=True)
    out_ref[4:5, :] += jnp.sum(l2_blk, axis=0, keepdims=True)

    def ce_stage(g, carry):
        acc5, accn = carry
        sl = pl.ds(g * 8, 8)
        valid = (idx8 + jnp.float32(1024.0) * g.astype(f32)) < jnp.float32(A)
        maxp = pconf_ref[0, 0, sl, :]
        for c in range(1, C):
            maxp = jnp.maximum(maxp, pconf_ref[0, c, sl, :])
        conf2 = conf2_s[sl, :]
        s = jnp.zeros((8, 128), f32)
        picked = jnp.zeros((8, 128), f32)
        for c in range(C):
            pc = pconf_ref[0, c, sl, :]
            s = s + jnp.exp(pc - maxp)
            picked = picked + jnp.where(conf2 == jnp.float32(c), pc, 0.0)
        ce_all = maxp + jnp.log(s) - picked
        pos2 = pos2_s[sl, :]
        neg_s[sl, :] = jnp.where((conf2 == 0.0) & (omask_s[sl, :] > 0.5) &
                                 valid, ce_all, -1.0)
        return acc5 + ce_all * pos2, accn + pos2

    (acc5, accn) = jax.lax.fori_loop(
        0, NG, ce_stage, (jnp.zeros((8, 128), f32), jnp.zeros((8, 128), f32)))
    out_ref[5:6, :] += jnp.sum(acc5, axis=0, keepdims=True)

    pos_num = jnp.sum(accn)
    k = jnp.maximum(10.0, jnp.minimum(pos_num * _NEG_POS,
                                      jnp.float32(A) - pos_num))

    def count_sum(thr):
        def body(g, carry):
            cacc, sacc = carry
            ng = neg_s[pl.ds(g * 8, 8), :]
            above = ng > thr
            return (cacc + jnp.where(above, 1.0, 0.0),
                    sacc + jnp.where(above, ng, 0.0))

        cacc, sacc = jax.lax.fori_loop(
            0, NG, body, (jnp.zeros((8, 128), f32), jnp.zeros((8, 128), f32)))
        return jnp.sum(cacc), jnp.sum(sacc)

    def bis(_, carry):
        lo, hi = carry
        mid = 0.5 * (lo + hi)

        def body(g, cacc):
            return cacc + jnp.where(neg_s[pl.ds(g * 8, 8), :] > mid, 1.0, 0.0)

        cnt = jnp.sum(jax.lax.fori_loop(0, NG, body,
                                        jnp.zeros((8, 128), f32)))
        ge = cnt >= k
        return jnp.where(ge, mid, lo), jnp.where(ge, hi, mid)

    lo, hi = jax.lax.fori_loop(0, 32, bis, (jnp.float32(-2.0),
                                            jnp.float32(64.0)))
    c_gt, s_gt = count_sum(jnp.maximum(lo, -0.5))
    mined = s_gt + jnp.where(lo > -0.5, (k - c_gt) * lo, 0.0)
    out_ref[6:7, :] += jnp.where(lane < 1.0, mined, 0.0)


def kernel(objectness, refine_loc, pred_conf, pred_loc, anchors, targets):
    B, A, C = pred_conf.shape
    T = targets.shape[1]
    AP = ((A + 1023) // 1024) * 1024
    R = AP // 128
    TP = ((T + 7) // 8) * 8

    priors = anchors[0]
    pad = jnp.concatenate([jnp.full((AP - A, 2), -10.0, jnp.float32),
                           jnp.ones((AP - A, 2), jnp.float32)], axis=1)
    priors_pl = jnp.concatenate([priors, pad], axis=0).T.reshape(4, R, 128)

    def planes(x):
        k = x.shape[-1]
        xp = jnp.pad(x, ((0, 0), (0, AP - A), (0, 0)))
        return xp.transpose(0, 2, 1).reshape(B, k, R, 128)

    obj_pl = planes(objectness)
    rloc_pl = planes(refine_loc)
    ploc_pl = planes(pred_loc)
    pconf_pl = planes(pred_conf)
    tgt = jnp.pad(targets, ((0, 0), (0, TP - T), (0, 3)))

    krn = functools.partial(_loss_kernel, A=A, C=C, TP=TP, R=R)
    out = pl.pallas_call(
        krn,
        grid=(B,),
        in_specs=[
            pl.BlockSpec((1, TP, 8), lambda b: (b, 0, 0)),
            pl.BlockSpec((4, R, 128), lambda b: (0, 0, 0)),
            pl.BlockSpec((1, 2, R, 128), lambda b: (b, 0, 0, 0)),
            pl.BlockSpec((1, 4, R, 128), lambda b: (b, 0, 0, 0)),
            pl.BlockSpec((1, 4, R, 128), lambda b: (b, 0, 0, 0)),
            pl.BlockSpec((1, C, R, 128), lambda b: (b, 0, 0, 0)),
        ],
        out_specs=pl.BlockSpec((8, 128), lambda b: (0, 0)),
        out_shape=jax.ShapeDtypeStruct((8, 128), jnp.float32),
        scratch_shapes=[pltpu.VMEM((R, 128), jnp.float32) for _ in range(7)],
        compiler_params=pltpu.CompilerParams(
            dimension_semantics=("arbitrary",)),
    )(tgt, priors_pl, obj_pl, rloc_pl, ploc_pl, pconf_pl)

    sums = jnp.sum(out, axis=1)
    arm_n, arm_loc, arm_cls, n, loc, cls_pos, neg_sum = (
        sums[0], sums[1], sums[2], sums[3], sums[4], sums[5], sums[6])
    class_loss = (cls_pos + neg_sum) / n
    loc_loss = loc / n
    arm_cls_loss = 0.04 * arm_cls / arm_n
    arm_loc_loss = arm_loc / arm_n
    total = class_loss + loc_loss + arm_cls_loss + arm_loc_loss
    return (total, class_loss, loc_loss, arm_cls_loss, arm_loc_loss)


# speedup vs baseline: 1.1455x; 1.0186x over previous
import functools

import jax
import jax.numpy as jnp
from jax.experimental import pallas as pl
from jax.experimental.pallas import tpu as pltpu

_MATCH_THRESH = 0.5
_NEG_POS = 3.0
_V0 = 0.1
_V1 = 0.2


def _smooth_l1(x):
    ax = jnp.abs(x)
    return jnp.where(ax < 1.0, 0.5 * ax * ax, ax - 0.5)


def _loss_kernel(tgt_ref, priors_ref, obj_ref, rloc_ref, ploc_ref, pconf_ref,
                 out_ref, bto_s, bti_s, pos1_s, omask_s, conf2_s, pos2_s, neg_s,
                 *, A, C, TP, R):
    b = pl.program_id(0)

    @pl.when(b == 0)
    def _init():
        out_ref[...] = jnp.zeros_like(out_ref)

    f32 = jnp.float32
    lane = jax.lax.broadcasted_iota(jnp.int32, (1, 128), 1).astype(f32)
    t_col = jax.lax.broadcasted_iota(jnp.int32, (TP, 1), 0).astype(f32)

    tg = tgt_ref[0]
    tx1 = tg[:, 0:1]
    ty1 = tg[:, 1:2]
    tx2 = tg[:, 2:3]
    ty2 = tg[:, 3:4]
    tlab = tg[:, 4:5]
    area_t = (tx2 - tx1) * (ty2 - ty1)

    NCH = min(64, R)

    def anchor_block(j4, refined):
        sl = pl.ds(j4, NCH)
        cx = priors_ref[0, sl, :]
        cy = priors_ref[1, sl, :]
        w = priors_ref[2, sl, :]
        h = priors_ref[3, sl, :]
        if refined:
            l0 = rloc_ref[0, 0, sl, :]
            l1 = rloc_ref[0, 1, sl, :]
            l2 = rloc_ref[0, 2, sl, :]
            l3 = rloc_ref[0, 3, sl, :]
            cx = cx + l0 * (_V0 * w)
            cy = cy + l1 * (_V0 * h)
            w = w * jnp.exp(l2 * _V1)
            h = h * jnp.exp(l3 * _V1)
        return cx, cy, w, h

    def overlaps_row(cx, cy, w, h, kk):
        cxk = cx[kk:kk + 1, :]
        cyk = cy[kk:kk + 1, :]
        wk = w[kk:kk + 1, :]
        hk = h[kk:kk + 1, :]
        ax1 = cxk - 0.5 * wk
        ay1 = cyk - 0.5 * hk
        ax2 = cxk + 0.5 * wk
        ay2 = cyk + 0.5 * hk
        iw = jnp.clip(jnp.minimum(tx2, ax2) - jnp.maximum(tx1, ax1),
                      0.0, None)
        ih = jnp.clip(jnp.minimum(ty2, ay2) - jnp.maximum(ty1, ay1),
                      0.0, None)
        inter = iw * ih
        area_a = (ax2 - ax1) * (ay2 - ay1)
        return inter / (area_t + area_a - inter)

    def pass_a(refined):
        def body(jj, carry):
            run_max, run_idx = carry
            j4 = jj * NCH
            cx, cy, w, h = anchor_block(j4, refined)
            base = jnp.float32(128.0) * j4.astype(f32) + lane
            btos, btis = [], []
            m_acc = i_acc = None
            for kk in range(0, NCH, 2):
                ov_a = overlaps_row(cx, cy, w, h, kk)
                ov_b = overlaps_row(cx, cy, w, h, kk + 1)
                for ov in (ov_a, ov_b):
                    btos.append(jnp.max(ov, axis=0, keepdims=True))
                    btis.append(jnp.min(jnp.where(ov == btos[-1], t_col, 1e9),
                                        axis=0, keepdims=True))
                m = jnp.maximum(ov_a, ov_b)
                i = jnp.where(ov_b > ov_a, base + jnp.float32(128.0 * (kk + 1)),
                              base + jnp.float32(128.0 * kk))
                if m_acc is None:
                    m_acc, i_acc = m, i
                else:
                    i_acc = jnp.where(m > m_acc, i, i_acc)
                    m_acc = jnp.maximum(m_acc, m)
            bto_s[pl.ds(j4, NCH), :] = jnp.concatenate(btos, axis=0)
            bti_s[pl.ds(j4, NCH), :] = jnp.concatenate(btis, axis=0)
            m, i = m_acc, i_acc
            better = m > run_max
            run_idx = jnp.where(better, i, run_idx)
            run_max = jnp.maximum(run_max, m)
            return run_max, run_idx

        init = (jnp.full((TP, 128), -1e9, f32), jnp.full((TP, 128), 1e9, f32))
        run_max, run_idx = jax.lax.fori_loop(0, R // NCH, body, init)
        gmax = jnp.max(run_max, axis=1, keepdims=True)
        bpidx = jnp.min(jnp.where(run_max == gmax, run_idx, 1e9),
                        axis=1, keepdims=True)
        return jnp.where(t_col < 50.0, bpidx, -1.0)

    def pass_b(refined, bpidx, loc_pred_ref, store_blk, masked):
        sub4 = jax.lax.broadcasted_iota(jnp.int32, (NCH, 128), 0).astype(f32)
        lane4 = jax.lax.broadcasted_iota(jnp.int32, (NCH, 128), 1).astype(f32)

        def body(jj, carry):
            acc_n, acc_l = carry
            j4 = jj * NCH
            sl = pl.ds(j4, NCH)
            bti_u = bti_s[sl, :]
            bto_u = bto_s[sl, :]
            rows = []
            for kk in range(NCH):
                ga = jnp.float32(128.0) * (j4.astype(f32) +
                                           jnp.float32(kk)) + lane
                eqf = bpidx == ga
                forced_t = jnp.max(jnp.where(eqf, t_col, -1.0), axis=0,
                                   keepdims=True)
                hasf = forced_t >= 0.0
                bti_f = jnp.where(hasf, forced_t, bti_u[kk:kk + 1, :])
                bto_f = jnp.where(hasf, 2.0, bto_u[kk:kk + 1, :])
                oh = t_col == bti_f
                gsum = lambda v: jnp.sum(jnp.where(oh, v, 0.0), axis=0,
                                         keepdims=True)
                rows.append((gsum(tlab), gsum(tx1), gsum(ty1), gsum(tx2),
                             gsum(ty2), bto_f))
            cat = lambda q: jnp.concatenate([r[q] for r in rows], axis=0)
            conf_g = cat(0)
            mx1 = cat(1)
            my1 = cat(2)
            mx2 = cat(3)
            my2 = cat(4)
            bto4 = cat(5)
            ga4 = (jnp.float32(128.0) * j4.astype(f32) + sub4 * 128.0 + lane4)
            conf = jnp.where(bto4 < _MATCH_THRESH, 0.0, conf_g)
            posf = jnp.where((conf > 0.0) & (ga4 < jnp.float32(A)), 1.0, 0.0)
            if masked:
                posf = posf * omask_s[sl, :]
            cx, cy, w, h = anchor_block(j4, refined)
            ecx = ((mx1 + mx2) * 0.5 - cx) / (_V0 * w)
            ecy = ((my1 + my2) * 0.5 - cy) / (_V0 * h)
            ew = jnp.log(jnp.maximum((mx2 - mx1) / w, 1e-8)) / _V1
            eh = jnp.log(jnp.maximum((my2 - my1) / h, 1e-8)) / _V1
            p0 = loc_pred_ref[0, 0, sl, :]
            p1 = loc_pred_ref[0, 1, sl, :]
            p2 = loc_pred_ref[0, 2, sl, :]
            p3 = loc_pred_ref[0, 3, sl, :]
            ll = (_smooth_l1(p0 - ecx) + _smooth_l1(p1 - ecy) +
                  _smooth_l1(p2 - ew) + _smooth_l1(p3 - eh))
            store_blk(sl, conf, posf)
            return acc_n + posf, acc_l + ll * posf

        init = (jnp.zeros((NCH, 128), f32), jnp.zeros((NCH, 128), f32))
        return jax.lax.fori_loop(0, R // NCH, body, init)

    bpidx1 = pass_a(False)

    def store1(sl, conf, posf):
        pos1_s[sl, :] = posf

    n1_blk, l1_blk = pass_b(False, bpidx1, rloc_ref, store1, False)
    out_ref[0:1, :] += jnp.sum(n1_blk, axis=0, keepdims=True)
    out_ref[1:2, :] += jnp.sum(l1_blk, axis=0, keepdims=True)

    NG = R // 8
    idx8 = (jax.lax.broadcasted_iota(jnp.int32, (8, 128), 0) * 128 +
            jax.lax.broadcasted_iota(jnp.int32, (8, 128), 1)).astype(f32)
    log_theta = jnp.log(jnp.float32(0.99))

    def obj_stage(g, acc):
        sl = pl.ds(g * 8, 8)
        valid = (idx8 + jnp.float32(1024.0) * g.astype(f32)) < jnp.float32(A)
        o0 = obj_ref[0, 0, sl, :]
        o1 = obj_ref[0, 1, sl, :]
        m = jnp.maximum(o0, o1)
        lse2 = m + jnp.log(jnp.exp(o0 - m) + jnp.exp(o1 - m))
        pos1 = pos1_s[sl, :]
        ce2 = lse2 - (o0 * (1.0 - pos1) + o1 * pos1)
        omask_s[sl, :] = jnp.where((o0 - lse2) < log_theta, 1.0, 0.0)
        return acc + jnp.where(valid, ce2, 0.0)

    acc2 = jax.lax.fori_loop(0, NG, obj_stage, jnp.zeros((8, 128), f32))
    out_ref[2:3, :] += jnp.sum(acc2, axis=0, keepdims=True)

    bpidx2 = pass_a(True)

    def store2(sl, conf, posf):
        conf2_s[sl, :] = conf
        pos2_s[sl, :] = posf

    n2_blk, l2_blk = pass_b(True, bpidx2, ploc_ref, store2, True)
    out_ref[3:4, :] += jnp.sum(n2_blk, axis=0, keepdims=True)
    out_ref[4:5, :] += jnp.sum(l2_blk, axis=0, keepdims=True)

    def ce_stage(g, carry):
        acc5, accn = carry
        sl = pl.ds(g * 8, 8)
        valid = (idx8 + jnp.float32(1024.0) * g.astype(f32)) < jnp.float32(A)
        maxp = pconf_ref[0, 0, sl, :]
        for c in range(1, C):
            maxp = jnp.maximum(maxp, pconf_ref[0, c, sl, :])
        conf2 = conf2_s[sl, :]
        s = jnp.zeros((8, 128), f32)
        picked = jnp.zeros((8, 128), f32)
        for c in range(C):
            pc = pconf_ref[0, c, sl, :]
            s = s + jnp.exp(pc - maxp)
            picked = picked + jnp.where(conf2 == jnp.float32(c), pc, 0.0)
        ce_all = maxp + jnp.log(s) - picked
        pos2 = pos2_s[sl, :]
        neg_s[sl, :] = jnp.where((conf2 == 0.0) & (omask_s[sl, :] > 0.5) &
                                 valid, ce_all, -1.0)
        return acc5 + ce_all * pos2, accn + pos2

    (acc5, accn) = jax.lax.fori_loop(
        0, NG, ce_stage, (jnp.zeros((8, 128), f32), jnp.zeros((8, 128), f32)))
    out_ref[5:6, :] += jnp.sum(acc5, axis=0, keepdims=True)

    pos_num = jnp.sum(accn)
    k = jnp.maximum(10.0, jnp.minimum(pos_num * _NEG_POS,
                                      jnp.float32(A) - pos_num))

    def count_sum(thr):
        def body(g, carry):
            cacc, sacc = carry
            ng = neg_s[pl.ds(g * 8, 8), :]
            above = ng > thr
            return (cacc + jnp.where(above, 1.0, 0.0),
                    sacc + jnp.where(above, ng, 0.0))

        cacc, sacc = jax.lax.fori_loop(
            0, NG, body, (jnp.zeros((8, 128), f32), jnp.zeros((8, 128), f32)))
        return jnp.sum(cacc), jnp.sum(sacc)

    def bis(_, carry):
        lo, hi = carry
        mid = 0.5 * (lo + hi)

        def body(g, cacc):
            return cacc + jnp.where(neg_s[pl.ds(g * 8, 8), :] > mid, 1.0, 0.0)

        cnt = jnp.sum(jax.lax.fori_loop(0, NG, body,
                                        jnp.zeros((8, 128), f32)))
        ge = cnt >= k
        return jnp.where(ge, mid, lo), jnp.where(ge, hi, mid)

    lo, hi = jax.lax.fori_loop(0, 32, bis, (jnp.float32(-2.0),
                                            jnp.float32(64.0)))
    c_gt, s_gt = count_sum(jnp.maximum(lo, -0.5))
    mined = s_gt + jnp.where(lo > -0.5, (k - c_gt) * lo, 0.0)
    out_ref[6:7, :] += jnp.where(lane < 1.0, mined, 0.0)


def kernel(objectness, refine_loc, pred_conf, pred_loc, anchors, targets):
    B, A, C = pred_conf.shape
    T = targets.shape[1]
    AP = ((A + 1023) // 1024) * 1024
    R = AP // 128
    TP = ((T + 7) // 8) * 8

    priors = anchors[0]
    pad = jnp.concatenate([jnp.full((AP - A, 2), -10.0, jnp.float32),
                           jnp.ones((AP - A, 2), jnp.float32)], axis=1)
    priors_pl = jnp.concatenate([priors, pad], axis=0).T.reshape(4, R, 128)

    def planes(x):
        k = x.shape[-1]
        xp = jnp.pad(x, ((0, 0), (0, AP - A), (0, 0)))
        return xp.transpose(0, 2, 1).reshape(B, k, R, 128)

    obj_pl = planes(objectness)
    rloc_pl = planes(refine_loc)
    ploc_pl = planes(pred_loc)
    pconf_pl = planes(pred_conf)
    tgt = jnp.pad(targets, ((0, 0), (0, TP - T), (0, 3)))

    krn = functools.partial(_loss_kernel, A=A, C=C, TP=TP, R=R)
    out = pl.pallas_call(
        krn,
        grid=(B,),
        in_specs=[
            pl.BlockSpec((1, TP, 8), lambda b: (b, 0, 0)),
            pl.BlockSpec((4, R, 128), lambda b: (0, 0, 0)),
            pl.BlockSpec((1, 2, R, 128), lambda b: (b, 0, 0, 0)),
            pl.BlockSpec((1, 4, R, 128), lambda b: (b, 0, 0, 0)),
            pl.BlockSpec((1, 4, R, 128), lambda b: (b, 0, 0, 0)),
            pl.BlockSpec((1, C, R, 128), lambda b: (b, 0, 0, 0)),
        ],
        out_specs=pl.BlockSpec((8, 128), lambda b: (0, 0)),
        out_shape=jax.ShapeDtypeStruct((8, 128), jnp.float32),
        scratch_shapes=[pltpu.VMEM((R, 128), jnp.float32) for _ in range(7)],
        compiler_params=pltpu.CompilerParams(
            dimension_semantics=("arbitrary",)),
    )(tgt, priors_pl, obj_pl, rloc_pl, ploc_pl, pconf_pl)

    sums = jnp.sum(out, axis=1)
    arm_n, arm_loc, arm_cls, n, loc, cls_pos, neg_sum = (
        sums[0], sums[1], sums[2], sums[3], sums[4], sums[5], sums[6])
    class_loss = (cls_pos + neg_sum) / n
    loc_loss = loc / n
    arm_cls_loss = 0.04 * arm_cls / arm_n
    arm_loc_loss = arm_loc / arm_n
    total = class_loss + loc_loss + arm_cls_loss + arm_loc_loss
    return (total, class_loss, loc_loss, arm_cls_loss, arm_loc_loss)
